# baseline (speedup 1.0000x reference)
import functools

import jax
import jax.numpy as jnp
from jax import lax
from jax.experimental import pallas as pl
from jax.experimental.pallas import tpu as pltpu
from jax.experimental.pallas import tpu_sc as plsc

L = 16
NC = 2
NS = 16
NW = NC * NS

B = 16384
D = 32
V = 1_000_000
VTAIL = V - (V // 128) * 128 + 64
TAILLO = 999936
SHARD = 31744
STRIDE = 31232
BLK = 1024
NBLK = SHARD // BLK
CHUNK = B // NW
SROWS = B + 8 * NW
CAP = B + L


def _stage_body(users_hbm, items_hbm, tab_hbms, tail_hbms, st_hbms,
                idx_v, bkt_v, cl_v,
                blk_a, blk_b, tail_v, slab_a, slab_b, pos_a, pos_b,
                sem_a, sem_b, sca, scb):
    wid = lax.axis_index("s") * NC + lax.axis_index("c")
    lo = wid * STRIDE
    lanes = lax.iota(jnp.int32, L)
    dummy = B + wid * 8 + (lanes & 7)
    is31 = wid == NW - 1

    for pos_x in (pos_a, pos_b):
        pos_x[0, pl.ds(0, L)] = dummy
        pos_x[0, pl.ds(L, L)] = dummy
    pltpu.async_copy(slab_a, st_hbms[0].at[pos_a.at[0]], sca)
    pltpu.async_copy(slab_b, st_hbms[0].at[pos_b.at[0]], scb)

    def drain_sc(slab_x, sc_x):
        pltpu.make_async_copy(st_hbms[0].at[pl.ds(0, 2 * L)], slab_x,
                              sc_x).wait()

    for t in range(2):
        src_idx = (users_hbm, items_hbm)[t]
        tab = tab_hbms[t]
        st = st_hbms[t]
        pltpu.sync_copy(src_idx, idx_v)
        pltpu.sync_copy(tail_hbms[t], tail_v)

        def scan(g, n):
            li = idx_v[pl.ds(g * L, L)] - lo
            m = (li >= 0) & (li < SHARD)
            m = m | ((li >= SHARD) & (li < SHARD + 64) & is31)
            packed = (li << 14) | (g * L + lanes)
            plsc.store_compressed(bkt_v.at[pl.ds(n, L)], packed, mask=m)
            return n + plsc.all_reduce_population_count(m)[0]

        n = lax.fori_loop(0, B // L, scan, 0)
        qmax = (n + L - 1) // L

        def filt(blo_rel, bhi_rel):
            def fscan(q, nb):
                valid = q * L + lanes < n
                packed = bkt_v[pl.ds(q * L, L)]
                li = packed >> 14
                m = valid & (li >= blo_rel) & (li < bhi_rel)
                plsc.store_compressed(cl_v.at[pl.ds(nb, L)], packed, mask=m)
                return nb + plsc.all_reduce_population_count(m)[0]
            return lax.fori_loop(0, qmax, fscan, 0)

        def build(src_ref, blo_rel, nb, bat, g, slab_x, pos_x):
            valid = bat * 2 * L + g * L + lanes < nb
            packed = cl_v[pl.ds(bat * 2 * L + g * L, L)]
            col = jnp.where(valid, (packed >> 14) - blo_rel, 0)
            pos_x[0, pl.ds(g * L, L)] = jnp.where(valid, packed & (B - 1),
                                                  dummy)
            rows = g * L + lanes
            for c in range(D):
                cc = jnp.full((L,), c, jnp.int32)
                if src_ref is tail_v:
                    flat = col * D + c
                    val = plsc.load_gather(src_ref, [flat >> 7, flat & 127])
                else:
                    val = plsc.load_gather(src_ref, [cc, col])
                plsc.store_scatter(slab_x, [rows, cc], val)

        def process(src_ref, b_rel, slab_x, pos_x, sc_x):
            nb = filt(b_rel, b_rel + BLK)

            @pl.when(nb > 0)
            def _():
                def batch(bat, carry):
                    drain_sc(slab_x, sc_x)
                    build(src_ref, b_rel, nb, bat, 0, slab_x, pos_x)
                    build(src_ref, b_rel, nb, bat, 1, slab_x, pos_x)
                    pltpu.async_copy(slab_x, st.at[pos_x.at[0]], sc_x)
                    return carry

                lax.fori_loop(0, (nb + 2 * L - 1) // (2 * L), batch, 0)

        def fire(b, buf, sem):
            return pltpu.async_copy(tab.at[:, pl.ds(lo + b * BLK, BLK)],
                                    buf, sem)

        def drain(buf, sem):
            pltpu.make_async_copy(tab.at[:, pl.ds(0, BLK)], buf, sem).wait()

        fire(0, blk_a, sem_a)

        def pair(p, carry):
            fire(2 * p + 1, blk_b, sem_b)
            drain(blk_a, sem_a)
            process(blk_a, 2 * p * BLK, slab_a, pos_a, sca)

            @pl.when(p < NBLK // 2 - 1)
            def _():
                fire(2 * p + 2, blk_a, sem_a)

            @pl.when(p == NBLK // 2 - 1)
            def _():
                fire(NBLK - 1, blk_a, sem_a)

            drain(blk_b, sem_b)
            process(blk_b, (2 * p + 1) * BLK, slab_b, pos_b, scb)
            return carry

        lax.fori_loop(0, NBLK // 2, pair, 0)
        drain(blk_a, sem_a)
        process(blk_a, (NBLK - 1) * BLK, slab_a, pos_a, sca)

        @pl.when(is31)
        def _():
            nb = filt(SHARD, SHARD + 64)

            @pl.when(nb > 0)
            def _():
                def batch(bat, carry):
                    drain_sc(slab_b, scb)
                    build(tail_v, SHARD, nb, bat, 0, slab_b, pos_b)
                    build(tail_v, SHARD, nb, bat, 1, slab_b, pos_b)
                    pltpu.async_copy(slab_b, st.at[pos_b.at[0]], scb)
                    return carry

                lax.fori_loop(0, (nb + 2 * L - 1) // (2 * L), batch, 0)

    drain_sc(slab_a, sca)
    drain_sc(slab_b, scb)


@functools.partial(
    pl.kernel,
    out_type=(jax.ShapeDtypeStruct((SROWS, 128), jnp.float32),
              jax.ShapeDtypeStruct((SROWS, 128), jnp.float32)),
    mesh=plsc.VectorSubcoreMesh(core_axis_name="c", subcore_axis_name="s"),
    scratch_types=[
        pltpu.VMEM((B,), jnp.int32),
        pltpu.VMEM((CAP,), jnp.int32),
        pltpu.VMEM((CAP,), jnp.int32),
        pltpu.VMEM((D, BLK), jnp.float32),
        pltpu.VMEM((D, BLK), jnp.float32),
        pltpu.VMEM((16, 128), jnp.float32),
        pltpu.VMEM((2 * L, 128), jnp.float32),
        pltpu.VMEM((2 * L, 128), jnp.float32),
        pltpu.VMEM((1, 2 * L), jnp.int32),
        pltpu.VMEM((1, 2 * L), jnp.int32),
        pltpu.SemaphoreType.DMA,
        pltpu.SemaphoreType.DMA,
        pltpu.SemaphoreType.DMA,
        pltpu.SemaphoreType.DMA,
    ],
    compiler_params=pltpu.CompilerParams(
        needs_layout_passes=False, use_tc_tiling_on_sc=True),
)
def _stage(users_hbm, items_hbm, utab, itab, utail, itail,
           st_u, st_i,
           idx_v, bkt_v, cl_v, blk_a, blk_b, tail_v,
           slab_a, slab_b, pos_a, pos_b,
           sem_a, sem_b, sca, scb):
    _stage_body(users_hbm, items_hbm, (utab, itab), (utail, itail),
                (st_u, st_i),
                idx_v, bkt_v, cl_v,
                blk_a, blk_b, tail_v, slab_a, slab_b, pos_a, pos_b,
                sem_a, sem_b, sca, scb)


def _dot_body(st_u, st_i, out_hbm, ubuf, ibuf, out_v, sem):
    wid = lax.axis_index("s") * NC + lax.axis_index("c")
    base = wid * CHUNK

    for h in range(2):
        hb = base + h * (CHUNK // 2)
        pltpu.sync_copy(st_u.at[pl.ds(hb, CHUNK // 2)], ubuf)
        pltpu.sync_copy(st_i.at[pl.ds(hb, CHUNK // 2)], ibuf)

        def group(g, carry):
            row = g * L + lax.iota(jnp.int32, L)
            acc = jnp.zeros((L,), jnp.float32)
            for d in range(D):
                col = jnp.full((L,), d, jnp.int32)
                u = plsc.load_gather(ubuf, [row, col])
                v = plsc.load_gather(ibuf, [row, col])
                acc = acc + u * v
            out_v[pl.ds(h * (CHUNK // 2) + g * L, L)] = acc
            return carry

        lax.fori_loop(0, CHUNK // 2 // L, group, 0)

    pltpu.sync_copy(out_v, out_hbm.at[pl.ds(base, CHUNK)])


@functools.partial(
    pl.kernel,
    out_type=jax.ShapeDtypeStruct((B,), jnp.float32),
    mesh=plsc.VectorSubcoreMesh(core_axis_name="c", subcore_axis_name="s"),
    scratch_types=[
        pltpu.VMEM((CHUNK // 2, 128), jnp.float32),
        pltpu.VMEM((CHUNK // 2, 128), jnp.float32),
        pltpu.VMEM((CHUNK,), jnp.float32),
        pltpu.SemaphoreType.DMA,
    ],
    compiler_params=pltpu.CompilerParams(
        needs_layout_passes=False, use_tc_tiling_on_sc=True),
)
def _dot(st_u, st_i, out_hbm, ubuf, ibuf, out_v, sem):
    _dot_body(st_u, st_i, out_hbm, ubuf, ibuf, out_v, sem)


def kernel(users, items, user_table, item_table):
    u = users.astype(jnp.int32)
    it = items.astype(jnp.int32)
    ut = user_table.T
    itb = item_table.T
    utail = user_table[TAILLO:].reshape(16, 128)
    itail = item_table[TAILLO:].reshape(16, 128)
    st_u, st_i = _stage(u, it, ut, itb, utail, itail)
    return _dot(st_u, st_i)
